# baseline (speedup 1.0000x reference)
import functools

import jax
import jax.numpy as jnp
from jax import lax
from jax.experimental import pallas as pl
from jax.experimental.pallas import tpu as pltpu
from jax.experimental.pallas import tpu_sc as plsc

_N_NODES = 100000
_N_EDGES = 6400000
_NC = 2
_NS = 16
_NW = _NC * _NS
_EPT = _N_EDGES // _NW
_CHUNK = 4000
_NCHUNK = _EPT // _CHUNK


def _newton_sqrt(x):
    i = lax.bitcast_convert_type(x, jnp.int32)
    r = lax.bitcast_convert_type(jnp.int32(0x5F3759DF) - (i >> 1), jnp.float32)
    for _ in range(3):
        r = r * (1.5 - 0.5 * x * r * r)
    return x * r


def _sc_body(rt_h, off_h, ii_h, jj_h, out_h, table_v, ii_v, jj_v, off_v, acc_v):
    wid = lax.axis_index("s") * _NC + lax.axis_index("c")
    base = wid * _EPT
    iota4 = lax.iota(jnp.int32, 16) * 4
    for p in range(3):
        pltpu.sync_copy(rt_h.at[pl.ds(p * _N_NODES, _N_NODES)], table_v)

        def chunk_body(k, _, p=p):
            cb = base + k * _CHUNK
            pltpu.sync_copy(ii_h.at[pl.ds(cb, _CHUNK)], ii_v)
            pltpu.sync_copy(jj_h.at[pl.ds(cb, _CHUNK)], jj_v)
            pltpu.sync_copy(off_h.at[pl.ds(4 * cb, 4 * _CHUNK)], off_v)
            if p > 0:
                pltpu.sync_copy(out_h.at[pl.ds(cb, _CHUNK)], acc_v)

            def vec_body(v, _, p=p):
                s = v * 16
                ii = ii_v[pl.ds(s, 16)]
                jj = jj_v[pl.ds(s, 16)]
                gi = plsc.load_gather(table_v, [ii])
                gj = plsc.load_gather(table_v, [jj])
                o = plsc.load_gather(off_v, [iota4 + (4 * s + p)])
                d = gj - gi + o
                sq = d * d
                if p == 0:
                    acc_v[pl.ds(s, 16)] = sq
                elif p == 1:
                    acc_v[pl.ds(s, 16)] = acc_v[pl.ds(s, 16)] + sq
                else:
                    acc_v[pl.ds(s, 16)] = _newton_sqrt(acc_v[pl.ds(s, 16)] + sq)
                return 0

            lax.fori_loop(0, _CHUNK // 16, vec_body, 0, unroll=2)
            pltpu.sync_copy(acc_v, out_h.at[pl.ds(cb, _CHUNK)])
            return 0

        lax.fori_loop(0, _NCHUNK, chunk_body, 0)


@jax.jit
def kernel(R, offsets, idx_i, idx_j):
    rt = R.T.reshape(-1)
    offt = jnp.pad(offsets, ((0, 0), (0, 1))).reshape(-1)
    mesh = plsc.VectorSubcoreMesh(core_axis_name="c", subcore_axis_name="s")
    f = pl.kernel(
        _sc_body,
        out_type=jax.ShapeDtypeStruct((_N_EDGES,), jnp.float32),
        mesh=mesh,
        compiler_params=pltpu.CompilerParams(needs_layout_passes=False),
        scratch_types=[
            pltpu.VMEM((_N_NODES,), jnp.float32),
            pltpu.VMEM((_CHUNK,), jnp.int32),
            pltpu.VMEM((_CHUNK,), jnp.int32),
            pltpu.VMEM((4 * _CHUNK,), jnp.float32),
            pltpu.VMEM((_CHUNK,), jnp.float32),
        ],
    )
    return f(rt, offt, idx_i.astype(jnp.int32), idx_j.astype(jnp.int32))


# speedup vs baseline: 8.7192x; 8.7192x over previous
import functools

import jax
import jax.numpy as jnp
from jax import lax
from jax.experimental import pallas as pl
from jax.experimental.pallas import tpu as pltpu
from jax.experimental.pallas import tpu_sc as plsc

_N_NODES = 100000
_N_EDGES = 6400000
_NC = 2
_NS = 16
_NW = _NC * _NS
_EPT = _N_EDGES // _NW
_CHUNK = 4000
_NCHUNK = _EPT // _CHUNK


def _newton_sqrt(x):
    i = lax.bitcast_convert_type(x, jnp.int32)
    r = lax.bitcast_convert_type(jnp.int32(0x5F3759DF) - (i >> 1), jnp.float32)
    for _ in range(3):
        r = r * (1.5 - 0.5 * x * r * r)
    return x * r


def _sc_body(rt_h, ox_h, oy_h, oz_h, ii_h, jj_h, out_h,
             table_v, ii_v, jj_v, off_v, acc_v):
    wid = lax.axis_index("s") * _NC + lax.axis_index("c")
    base = wid * _EPT
    off_hs = (ox_h, oy_h, oz_h)
    for p in range(3):
        pltpu.sync_copy(rt_h.at[pl.ds(p * _N_NODES, _N_NODES)], table_v)

        def chunk_body(k, _, p=p):
            cb = base + k * _CHUNK
            pltpu.sync_copy(ii_h.at[pl.ds(cb, _CHUNK)], ii_v)
            pltpu.sync_copy(jj_h.at[pl.ds(cb, _CHUNK)], jj_v)
            pltpu.sync_copy(off_hs[p].at[pl.ds(cb, _CHUNK)], off_v)
            if p > 0:
                pltpu.sync_copy(out_h.at[pl.ds(cb, _CHUNK)], acc_v)

            def vec_body(v, _, p=p):
                s = v * 16
                ii = ii_v[pl.ds(s, 16)]
                jj = jj_v[pl.ds(s, 16)]
                gi = plsc.load_gather(table_v, [ii])
                gj = plsc.load_gather(table_v, [jj])
                d = gj - gi + off_v[pl.ds(s, 16)]
                sq = d * d
                if p == 0:
                    acc_v[pl.ds(s, 16)] = sq
                elif p == 1:
                    acc_v[pl.ds(s, 16)] = acc_v[pl.ds(s, 16)] + sq
                else:
                    acc_v[pl.ds(s, 16)] = _newton_sqrt(acc_v[pl.ds(s, 16)] + sq)
                return 0

            lax.fori_loop(0, _CHUNK // 16, vec_body, 0, unroll=2)
            pltpu.sync_copy(acc_v, out_h.at[pl.ds(cb, _CHUNK)])
            return 0

        lax.fori_loop(0, _NCHUNK, chunk_body, 0)


@jax.jit
def kernel(R, offsets, idx_i, idx_j):
    rt = R.T.reshape(-1)
    ox, oy, oz = offsets[:, 0], offsets[:, 1], offsets[:, 2]
    mesh = plsc.VectorSubcoreMesh(core_axis_name="c", subcore_axis_name="s")
    f = pl.kernel(
        _sc_body,
        out_type=jax.ShapeDtypeStruct((_N_EDGES,), jnp.float32),
        mesh=mesh,
        compiler_params=pltpu.CompilerParams(needs_layout_passes=False),
        scratch_types=[
            pltpu.VMEM((_N_NODES,), jnp.float32),
            pltpu.VMEM((_CHUNK,), jnp.int32),
            pltpu.VMEM((_CHUNK,), jnp.int32),
            pltpu.VMEM((_CHUNK,), jnp.float32),
            pltpu.VMEM((_CHUNK,), jnp.float32),
        ],
    )
    return f(rt, ox, oy, oz, idx_i.astype(jnp.int32), idx_j.astype(jnp.int32))


# speedup vs baseline: 8.7408x; 1.0025x over previous
import functools

import jax
import jax.numpy as jnp
from jax import lax
from jax.experimental import pallas as pl
from jax.experimental.pallas import tpu as pltpu
from jax.experimental.pallas import tpu_sc as plsc

_N_NODES = 100000
_N_EDGES = 6400000
_NC = 2
_NS = 16
_NW = _NC * _NS
_EPT = _N_EDGES // _NW
_CHUNK = 4000
_NCHUNK = _EPT // _CHUNK


def _newton_sqrt(x):
    i = lax.bitcast_convert_type(x, jnp.int32)
    r = lax.bitcast_convert_type(jnp.int32(0x5F3759DF) - (i >> 1), jnp.float32)
    for _ in range(3):
        r = r * (1.5 - 0.5 * x * r * r)
    return x * r


def _sc_body(rt_h, ox_h, oy_h, oz_h, ii_h, jj_h, out_h,
             table_v, ii_v, jj_v, off_v, acc_v):
    wid = lax.axis_index("s") * _NC + lax.axis_index("c")
    base = wid * _EPT
    off_hs = (ox_h, oy_h, oz_h)
    for p in range(3):
        pltpu.sync_copy(rt_h.at[pl.ds(p * _N_NODES, _N_NODES)], table_v)

        def chunk_body(k, _, p=p):
            cb = base + k * _CHUNK
            pltpu.sync_copy(ii_h.at[pl.ds(cb, _CHUNK)], ii_v)
            pltpu.sync_copy(jj_h.at[pl.ds(cb, _CHUNK)], jj_v)
            pltpu.sync_copy(off_hs[p].at[pl.ds(cb, _CHUNK)], off_v)
            if p > 0:
                pltpu.sync_copy(out_h.at[pl.ds(cb, _CHUNK)], acc_v)

            def vec_body(v, _, p=p):
                s = v * 16
                ii = ii_v[pl.ds(s, 16)]
                jj = jj_v[pl.ds(s, 16)]
                gi = plsc.load_gather(table_v, [ii])
                gj = plsc.load_gather(table_v, [jj])
                d = gj - gi + off_v[pl.ds(s, 16)]
                sq = d * d
                if p == 0:
                    acc_v[pl.ds(s, 16)] = sq
                elif p == 1:
                    acc_v[pl.ds(s, 16)] = acc_v[pl.ds(s, 16)] + sq
                else:
                    acc_v[pl.ds(s, 16)] = _newton_sqrt(acc_v[pl.ds(s, 16)] + sq)
                return 0

            lax.fori_loop(0, _CHUNK // 16, vec_body, 0, unroll=8)
            pltpu.sync_copy(acc_v, out_h.at[pl.ds(cb, _CHUNK)])
            return 0

        lax.fori_loop(0, _NCHUNK, chunk_body, 0)


@jax.jit
def kernel(R, offsets, idx_i, idx_j):
    rt = R.T.reshape(-1)
    ox, oy, oz = offsets[:, 0], offsets[:, 1], offsets[:, 2]
    mesh = plsc.VectorSubcoreMesh(core_axis_name="c", subcore_axis_name="s")
    f = pl.kernel(
        _sc_body,
        out_type=jax.ShapeDtypeStruct((_N_EDGES,), jnp.float32),
        mesh=mesh,
        compiler_params=pltpu.CompilerParams(needs_layout_passes=False),
        scratch_types=[
            pltpu.VMEM((_N_NODES,), jnp.float32),
            pltpu.VMEM((_CHUNK,), jnp.int32),
            pltpu.VMEM((_CHUNK,), jnp.int32),
            pltpu.VMEM((_CHUNK,), jnp.float32),
            pltpu.VMEM((_CHUNK,), jnp.float32),
        ],
    )
    return f(rt, ox, oy, oz, idx_i.astype(jnp.int32), idx_j.astype(jnp.int32))


# speedup vs baseline: 12.8881x; 1.4745x over previous
import jax
import jax.numpy as jnp
from jax import lax
from jax.experimental import pallas as pl
from jax.experimental.pallas import tpu as pltpu
from jax.experimental.pallas import tpu_sc as plsc

_N_NODES = 100000
_N_EDGES = 6400000
_NC = 2
_NS = 16
_NW = _NC * _NS
_EPT = _N_EDGES // _NW
_CHUNK = 2000
_NCHUNK = _EPT // _CHUNK
_GROUPS = _CHUNK // 16


def _newton_sqrt(x):
    i = lax.bitcast_convert_type(x, jnp.int32)
    r = lax.bitcast_convert_type(jnp.int32(0x5F3759DF) - (i >> 1), jnp.float32)
    for _ in range(3):
        r = r * (1.5 - 0.5 * x * r * r)
    return x * r


def _sc_body(rt_h, ox_h, oy_h, oz_h, ii_h, jj_h, out_h,
             table_v,
             ii0, ii1, jj0, jj1, of0, of1,
             ac0, ac1, ac2, ac3,
             is0, is1, js0, js1, os0, os1,
             as0, as1, as2, as3,
             ws0, ws1, ws2, ws3):
    wid = lax.axis_index("s") * _NC + lax.axis_index("c")
    base = wid * _EPT
    off_hs = (ox_h, oy_h, oz_h)
    ii_v, jj_v, of_v = (ii0, ii1), (jj0, jj1), (of0, of1)
    ac_v = (ac0, ac1, ac2, ac3)
    i_s, j_s, o_s = (is0, is1), (js0, js1), (os0, os1)
    a_s = (as0, as1, as2, as3)
    w_s = (ws0, ws1, ws2, ws3)

    def in_slice(h, k):
        return h.at[pl.ds(base + k * _CHUNK, _CHUNK)]

    def issue_inputs(p, k, sl):
        pltpu.async_copy(in_slice(ii_h, k), ii_v[sl], i_s[sl])
        pltpu.async_copy(in_slice(jj_h, k), jj_v[sl], j_s[sl])
        pltpu.async_copy(in_slice(off_hs[p], k), of_v[sl], o_s[sl])

    def wait_write(asl):
        pltpu.make_async_copy(ac_v[asl], in_slice(out_h, 0), w_s[asl]).wait()

    def issue_acc_read(k, asl):
        pltpu.async_copy(in_slice(out_h, k), ac_v[asl], a_s[asl])

    def wait_inputs(p, sl, asl):
        pltpu.make_async_copy(in_slice(ii_h, 0), ii_v[sl], i_s[sl]).wait()
        pltpu.make_async_copy(in_slice(jj_h, 0), jj_v[sl], j_s[sl]).wait()
        pltpu.make_async_copy(in_slice(off_hs[p], 0), of_v[sl], o_s[sl]).wait()
        if p > 0:
            pltpu.make_async_copy(in_slice(out_h, 0), ac_v[asl], a_s[asl]).wait()

    def compute(p, sl, asl):
        ii_b, jj_b, of_b, ac_b = ii_v[sl], jj_v[sl], of_v[sl], ac_v[asl]

        def vec_body(v, _):
            s = v * 16
            ii = ii_b[pl.ds(s, 16)]
            jj = jj_b[pl.ds(s, 16)]
            gi = plsc.load_gather(table_v, [ii])
            gj = plsc.load_gather(table_v, [jj])
            d = gj - gi + of_b[pl.ds(s, 16)]
            sq = d * d
            if p == 0:
                ac_b[pl.ds(s, 16)] = sq
            elif p == 1:
                ac_b[pl.ds(s, 16)] = ac_b[pl.ds(s, 16)] + sq
            else:
                ac_b[pl.ds(s, 16)] = _newton_sqrt(ac_b[pl.ds(s, 16)] + sq)
            return 0

        lax.fori_loop(0, _GROUPS, vec_body, 0, unroll=5)

    def iteration(p, k, b4, first_round):
        sl = b4 % 2
        nsl = 1 - sl
        nb4 = (b4 + 1) % 4

        def issue_next():
            issue_inputs(p, k + 1, nsl)
            if p > 0:
                if not (first_round and b4 < 3):
                    wait_write(nb4)
                issue_acc_read(k + 1, nb4)

        if first_round:
            issue_next()
        else:
            @pl.when(k + 1 < _NCHUNK)
            def _():
                issue_next()

        wait_inputs(p, sl, b4)
        if p == 0 and not first_round:
            wait_write(b4)
        compute(p, sl, b4)
        pltpu.async_copy(ac_v[b4], in_slice(out_h, k), w_s[b4])

    for p in range(3):
        pltpu.sync_copy(rt_h.at[pl.ds(p * _N_NODES, _N_NODES)], table_v)
        issue_inputs(p, 0, 0)
        if p > 0:
            issue_acc_read(0, 0)
        for b4 in range(4):
            iteration(p, b4, b4, first_round=True)

        def chunk_body(k4, _, p=p):
            for b4 in range(4):
                iteration(p, k4 * 4 + b4, b4, first_round=False)
            return 0

        lax.fori_loop(1, _NCHUNK // 4, chunk_body, 0)
        for b4 in range(4):
            wait_write(b4)


@jax.jit
def kernel(R, offsets, idx_i, idx_j):
    rt = R.T.reshape(-1)
    ox, oy, oz = offsets[:, 0], offsets[:, 1], offsets[:, 2]
    mesh = plsc.VectorSubcoreMesh(core_axis_name="c", subcore_axis_name="s")
    f = pl.kernel(
        _sc_body,
        out_type=jax.ShapeDtypeStruct((_N_EDGES,), jnp.float32),
        mesh=mesh,
        compiler_params=pltpu.CompilerParams(needs_layout_passes=False),
        scratch_types=[
            pltpu.VMEM((_N_NODES,), jnp.float32),
            pltpu.VMEM((_CHUNK,), jnp.int32),
            pltpu.VMEM((_CHUNK,), jnp.int32),
            pltpu.VMEM((_CHUNK,), jnp.int32),
            pltpu.VMEM((_CHUNK,), jnp.int32),
            pltpu.VMEM((_CHUNK,), jnp.float32),
            pltpu.VMEM((_CHUNK,), jnp.float32),
            pltpu.VMEM((_CHUNK,), jnp.float32),
            pltpu.VMEM((_CHUNK,), jnp.float32),
            pltpu.VMEM((_CHUNK,), jnp.float32),
            pltpu.VMEM((_CHUNK,), jnp.float32),
            pltpu.SemaphoreType.DMA,
            pltpu.SemaphoreType.DMA,
            pltpu.SemaphoreType.DMA,
            pltpu.SemaphoreType.DMA,
            pltpu.SemaphoreType.DMA,
            pltpu.SemaphoreType.DMA,
            pltpu.SemaphoreType.DMA,
            pltpu.SemaphoreType.DMA,
            pltpu.SemaphoreType.DMA,
            pltpu.SemaphoreType.DMA,
            pltpu.SemaphoreType.DMA,
            pltpu.SemaphoreType.DMA,
            pltpu.SemaphoreType.DMA,
            pltpu.SemaphoreType.DMA,
        ],
    )
    return f(rt, ox, oy, oz, idx_i.astype(jnp.int32), idx_j.astype(jnp.int32))
